# baseline (speedup 1.0000x reference)
import functools

import jax
import jax.numpy as jnp
from jax.experimental import pallas as pl

B = 128
V = 100000
N = V - 1
ROWS = 8

_SEED = 42
_KS0 = 0
_KS1 = _SEED
_KS2 = _KS0 ^ _KS1 ^ 0x1BD11BDA
_KS = (_KS0, _KS1, _KS2)
_ROT = ((13, 15, 26, 6), (17, 29, 16, 24))
import numpy as np

_TINY = np.float32(np.finfo(np.float32).tiny)


def _threefry_fold(ctr):
    u32 = lambda c: jnp.uint32(c & 0xFFFFFFFF)
    x0 = jnp.zeros_like(ctr) + u32(_KS0)
    x1 = ctr + u32(_KS1)
    for i in range(5):
        for r in _ROT[i % 2]:
            x0 = x0 + x1
            x1 = (x1 << jnp.uint32(r)) | (x1 >> jnp.uint32(32 - r))
            x1 = x1 ^ x0
        x0 = x0 + u32(_KS[(i + 1) % 3])
        x1 = x1 + u32(_KS[(i + 2) % 3] + (i + 1))
    return x0 ^ x1


def _gate_kernel(x_ref, g_ref):
    step = pl.program_id(0)
    row0 = step * ROWS

    rows = jax.lax.broadcasted_iota(jnp.int32, (ROWS, V), 0)
    cols = jax.lax.broadcasted_iota(jnp.int32, (ROWS, V), 1)
    flat = (row0 + rows) * V + cols

    bits = _threefry_fold(flat.astype(jnp.uint32))
    f = jax.lax.bitcast_convert_type(
        (bits >> jnp.uint32(9)) | jnp.uint32(0x3F800000), jnp.float32
    ) - jnp.float32(1.0)
    u = jnp.maximum(_TINY, f * (jnp.float32(1.0) - _TINY) + _TINY)
    s = -jnp.log(-jnp.log(u)) + jnp.log(x_ref[...] + jnp.float32(1e-20))

    m = jnp.max(s, axis=1, keepdims=True)
    big = jnp.int32(0x7FFFFFFF)
    c = jnp.min(jnp.where(s == m, cols, big), axis=1, keepdims=True)

    g_ref[...] = (cols[:, :N] < c).astype(jnp.float32)


@jax.jit
def kernel(x):
    g = pl.pallas_call(
        _gate_kernel,
        grid=(B // ROWS,),
        in_specs=[pl.BlockSpec((ROWS, V), lambda i: (i, 0))],
        out_specs=pl.BlockSpec((ROWS, N), lambda i: (i, 0)),
        out_shape=jax.ShapeDtypeStruct((B, N), jnp.float32),
    )(x)
    return (g, x)


# speedup vs baseline: 1.3150x; 1.3150x over previous
import jax
import jax.numpy as jnp
import numpy as np
from jax.experimental import pallas as pl

B = 128
V = 100000
N = V - 1
ROWS = 8
CHUNK = 1024
NFULL = V // CHUNK
TAIL0 = NFULL * CHUNK
TAIL = V - TAIL0

_SEED = 42
_KS0 = 0
_KS1 = _SEED
_KS2 = _KS0 ^ _KS1 ^ 0x1BD11BDA
_KS = (_KS0, _KS1, _KS2)
_ROT = ((13, 15, 26, 6), (17, 29, 16, 24))
_TINY = np.float32(np.finfo(np.float32).tiny)
_NEG_INF = np.float32(-np.inf)


def _threefry_fold(ctr):
    u32 = lambda c: jnp.uint32(c & 0xFFFFFFFF)
    x0 = ctr * jnp.uint32(0)
    x1 = ctr + u32(_KS1)
    for i in range(5):
        for r in _ROT[i % 2]:
            x0 = x0 + x1
            x1 = (x1 << jnp.uint32(r)) | (x1 >> jnp.uint32(32 - r))
            x1 = x1 ^ x0
        if _KS[(i + 1) % 3]:
            x0 = x0 + u32(_KS[(i + 1) % 3])
        x1 = x1 + u32(_KS[(i + 2) % 3] + (i + 1))
    return x0 ^ x1


def _score(xx, flat_ctr):
    bits = _threefry_fold(flat_ctr.astype(jnp.uint32))
    f = jax.lax.bitcast_convert_type(
        (bits >> jnp.uint32(9)) | jnp.uint32(0x3F800000), jnp.float32
    ) - jnp.float32(1.0)
    u = jnp.maximum(_TINY, f + _TINY)
    return (xx + jnp.float32(1e-20)) / (-jnp.log(u))


def _gate_kernel(x_ref, g_ref, xo_ref):
    step = pl.program_id(0)
    row0 = step * ROWS

    sub = jax.lax.broadcasted_iota(jnp.int32, (ROWS, CHUNK), 0)
    lane = jax.lax.broadcasted_iota(jnp.int32, (ROWS, CHUNK), 1)
    base = (row0 + sub) * V + lane

    def body(j, carry):
        m, idx = carry
        start = j * CHUNK
        r = _score(x_ref[:, pl.ds(start, CHUNK)], base + start)
        col = lane + start
        upd = r > m
        m = jnp.where(upd, r, m)
        idx = jnp.where(upd, col, idx)
        return m, idx

    m0 = jnp.full((ROWS, CHUNK), _NEG_INF, jnp.float32)
    i0 = jnp.zeros((ROWS, CHUNK), jnp.int32)
    m, idx = jax.lax.fori_loop(0, NFULL, body, (m0, i0))

    sub_t = jax.lax.broadcasted_iota(jnp.int32, (ROWS, TAIL), 0)
    lane_t = jax.lax.broadcasted_iota(jnp.int32, (ROWS, TAIL), 1)
    r_t = _score(x_ref[:, TAIL0:V], (row0 + sub_t) * V + lane_t + TAIL0)

    m1 = jnp.max(m, axis=1, keepdims=True)
    m2 = jnp.max(r_t, axis=1, keepdims=True)
    mrow = jnp.maximum(m1, m2)
    c1 = jnp.min(jnp.where(m == mrow, idx, jnp.int32(V)), axis=1, keepdims=True)
    c2 = jnp.min(jnp.where(r_t == mrow, lane_t + TAIL0, jnp.int32(V)),
                 axis=1, keepdims=True)
    c = jnp.minimum(c1, c2)

    cols = jax.lax.broadcasted_iota(jnp.int32, (ROWS, N), 1)
    g_ref[...] = (cols < c).astype(jnp.float32)
    xo_ref[...] = x_ref[...]


@jax.jit
def kernel(x):
    g, xo = pl.pallas_call(
        _gate_kernel,
        grid=(B // ROWS,),
        in_specs=[pl.BlockSpec((ROWS, V), lambda i: (i, 0))],
        out_specs=[
            pl.BlockSpec((ROWS, N), lambda i: (i, 0)),
            pl.BlockSpec((ROWS, V), lambda i: (i, 0)),
        ],
        out_shape=[
            jax.ShapeDtypeStruct((B, N), jnp.float32),
            jax.ShapeDtypeStruct((B, V), jnp.float32),
        ],
    )(x)
    return (g, xo)


# speedup vs baseline: 1.3662x; 1.0389x over previous
import jax
import jax.numpy as jnp
import numpy as np
from jax.experimental import pallas as pl

B = 128
V = 100000
N = V - 1
ROWS = 8
CHUNK = 1024
NPAIR = V // (2 * CHUNK)
TAIL0 = NPAIR * 2 * CHUNK
TAIL = V - TAIL0

_SEED = 42
_KS0 = 0
_KS1 = _SEED
_KS2 = _KS0 ^ _KS1 ^ 0x1BD11BDA
_KS = (_KS0, _KS1, _KS2)
_ROT = ((13, 15, 26, 6), (17, 29, 16, 24))
_TINY = np.float32(np.finfo(np.float32).tiny)
_NEG_INF = np.float32(-np.inf)


def _score(xx, x1):
    u32 = lambda c: jnp.uint32(c & 0xFFFFFFFF)
    x0 = x1
    for i in range(5):
        for j, r in enumerate(_ROT[i % 2]):
            if not (i == 0 and j == 0):
                x0 = x0 + x1
            x1 = (x1 << jnp.uint32(r)) | (x1 >> jnp.uint32(32 - r))
            x1 = x1 ^ x0
        if _KS[(i + 1) % 3]:
            x0 = x0 + u32(_KS[(i + 1) % 3])
        x1 = x1 + u32(_KS[(i + 2) % 3] + (i + 1))
    bits = x0 ^ x1
    f = jax.lax.bitcast_convert_type(
        (bits >> jnp.uint32(9)) | jnp.uint32(0x3F800000), jnp.float32
    ) - jnp.float32(1.0)
    u = jnp.maximum(_TINY, f + _TINY)
    return -jnp.log(-jnp.log(u)) + jnp.log(xx + jnp.float32(1e-20))


def _gate_kernel(x_ref, g_ref, xo_ref):
    step = pl.program_id(0)
    row0 = step * ROWS

    sub = jax.lax.broadcasted_iota(jnp.int32, (ROWS, CHUNK), 0)
    lane = jax.lax.broadcasted_iota(jnp.int32, (ROWS, CHUNK), 1)
    base42 = (row0 + sub) * V + lane + 42

    def body(j, carry):
        m, idx = carry
        sa = j * (2 * CHUNK)
        sb = sa + CHUNK
        ra = _score(x_ref[:, pl.ds(sa, CHUNK)], (base42 + sa).astype(jnp.uint32))
        rb = _score(x_ref[:, pl.ds(sb, CHUNK)], (base42 + sb).astype(jnp.uint32))
        ca = 2 * j
        upd = ra > m
        m = jnp.where(upd, ra, m)
        idx = jnp.where(upd, ca, idx)
        upd = rb > m
        m = jnp.where(upd, rb, m)
        idx = jnp.where(upd, ca + 1, idx)
        return m, idx

    m0 = jnp.full((ROWS, CHUNK), _NEG_INF, jnp.float32)
    i0 = jnp.zeros((ROWS, CHUNK), jnp.int32)
    m, idx = jax.lax.fori_loop(0, NPAIR, body, (m0, i0))

    sub_t = jax.lax.broadcasted_iota(jnp.int32, (ROWS, TAIL), 0)
    lane_t = jax.lax.broadcasted_iota(jnp.int32, (ROWS, TAIL), 1)
    r_t = _score(x_ref[:, TAIL0:V],
                 ((row0 + sub_t) * V + lane_t + (TAIL0 + 42)).astype(jnp.uint32))

    m1 = jnp.max(m, axis=1, keepdims=True)
    m2 = jnp.max(r_t, axis=1, keepdims=True)
    mrow = jnp.maximum(m1, m2)
    col = idx * CHUNK + lane
    c1 = jnp.min(jnp.where(m == mrow, col, jnp.int32(V)), axis=1, keepdims=True)
    c2 = jnp.min(jnp.where(r_t == mrow, lane_t + TAIL0, jnp.int32(V)),
                 axis=1, keepdims=True)
    c = jnp.minimum(c1, c2)

    cols = jax.lax.broadcasted_iota(jnp.int32, (ROWS, N), 1)
    g_ref[...] = (cols < c).astype(jnp.float32)
    xo_ref[...] = x_ref[...]


@jax.jit
def kernel(x):
    g, xo = pl.pallas_call(
        _gate_kernel,
        grid=(B // ROWS,),
        in_specs=[pl.BlockSpec((ROWS, V), lambda i: (i, 0))],
        out_specs=[
            pl.BlockSpec((ROWS, N), lambda i: (i, 0)),
            pl.BlockSpec((ROWS, V), lambda i: (i, 0)),
        ],
        out_shape=[
            jax.ShapeDtypeStruct((B, N), jnp.float32),
            jax.ShapeDtypeStruct((B, V), jnp.float32),
        ],
    )(x)
    return (g, xo)


# speedup vs baseline: 1.9888x; 1.4557x over previous
import jax
import jax.numpy as jnp
import numpy as np
from jax.experimental import pallas as pl
from jax.experimental.pallas import tpu as pltpu

B = 128
V = 100000
N = V - 1

VB = 4096
NVB = (V + VB - 1) // VB
CH = 64
NPAIR = VB // (2 * CH)

NGB = (N + VB - 1) // VB

_SEED = 42
_KS0 = 0
_KS1 = _SEED
_KS2 = _KS0 ^ _KS1 ^ 0x1BD11BDA
_KS = (_KS0, _KS1, _KS2)
_ROT = ((13, 15, 26, 6), (17, 29, 16, 24))
_TINY = np.float32(np.finfo(np.float32).tiny)
_NEG_INF = np.float32(-np.inf)


def _score(xx, x1):
    u32 = lambda c: jnp.uint32(c & 0xFFFFFFFF)
    x0 = x1
    for i in range(5):
        for j, r in enumerate(_ROT[i % 2]):
            if not (i == 0 and j == 0):
                x0 = x0 + x1
            x1 = (x1 << jnp.uint32(r)) | (x1 >> jnp.uint32(32 - r))
            x1 = x1 ^ x0
        if _KS[(i + 1) % 3]:
            x0 = x0 + u32(_KS[(i + 1) % 3])
        x1 = x1 + u32(_KS[(i + 2) % 3] + (i + 1))
    bits = x0 ^ x1
    f = jax.lax.bitcast_convert_type(
        (bits >> jnp.uint32(9)) | jnp.uint32(0x3F800000), jnp.float32
    ) - jnp.float32(1.0)
    u = jnp.maximum(_TINY, f + _TINY)
    return -jnp.log(-jnp.log(u)) + jnp.log(xx + jnp.float32(1e-20))


def _scan_kernel(xt_ref, c_ref, xo_ref, m_ref, idx_ref):
    step = pl.program_id(0)

    @pl.when(step == 0)
    def _init():
        m_ref[...] = jnp.full((CH, B), _NEG_INF, jnp.float32)
        idx_ref[...] = jnp.zeros((CH, B), jnp.int32)

    sub = jax.lax.broadcasted_iota(jnp.int32, (CH, B), 0)
    lane = jax.lax.broadcasted_iota(jnp.int32, (CH, B), 1)
    base42 = lane * V + sub + (step * VB + 42)
    v0 = step * VB

    def body(j, carry):
        m, idx = carry
        sa = j * (2 * CH)
        sb = sa + CH
        ra = _score(xt_ref[pl.ds(sa, CH), :], (base42 + sa).astype(jnp.uint32))
        rb = _score(xt_ref[pl.ds(sb, CH), :], (base42 + sb).astype(jnp.uint32))
        ra = jnp.where(v0 + sa + sub < V, ra, _NEG_INF)
        rb = jnp.where(v0 + sb + sub < V, rb, _NEG_INF)
        ca = step * (2 * NPAIR) + 2 * j
        upd = ra > m
        m = jnp.where(upd, ra, m)
        idx = jnp.where(upd, ca, idx)
        upd = rb > m
        m = jnp.where(upd, rb, m)
        idx = jnp.where(upd, ca + 1, idx)
        return m, idx

    m, idx = jax.lax.fori_loop(0, NPAIR, body, (m_ref[...], idx_ref[...]))
    m_ref[...] = m
    idx_ref[...] = idx

    xo_ref[...] = xt_ref[...]

    @pl.when(step == NVB - 1)
    def _finish():
        mm = m_ref[...]
        col = idx_ref[...] * CH + sub
        mrow = jnp.max(mm, axis=0, keepdims=True)
        c = jnp.min(jnp.where(mm == mrow, col, jnp.int32(V)),
                    axis=0, keepdims=True)
        c_ref[...] = jnp.broadcast_to(c, (8, B))


def _gate_kernel(c_ref, g_ref):
    step = pl.program_id(0)
    sub = jax.lax.broadcasted_iota(jnp.int32, (VB, B), 0)
    g_ref[...] = (step * VB + sub < c_ref[0:1, :]).astype(jnp.float32)


@jax.jit
def kernel(x):
    xt = x.T
    c8, xot = pl.pallas_call(
        _scan_kernel,
        grid=(NVB,),
        in_specs=[pl.BlockSpec((VB, B), lambda i: (i, 0))],
        out_specs=[
            pl.BlockSpec((8, B), lambda i: (0, 0)),
            pl.BlockSpec((VB, B), lambda i: (i, 0)),
        ],
        out_shape=[
            jax.ShapeDtypeStruct((8, B), jnp.int32),
            jax.ShapeDtypeStruct((V, B), jnp.float32),
        ],
        scratch_shapes=[
            pltpu.VMEM((CH, B), jnp.float32),
            pltpu.VMEM((CH, B), jnp.int32),
        ],
    )(xt)
    gt = pl.pallas_call(
        _gate_kernel,
        grid=(NGB,),
        in_specs=[pl.BlockSpec((8, B), lambda i: (0, 0))],
        out_specs=pl.BlockSpec((VB, B), lambda i: (i, 0)),
        out_shape=jax.ShapeDtypeStruct((N, B), jnp.float32),
    )(c8)
    return (gt.T, xot.T)


# speedup vs baseline: 2.0439x; 1.0277x over previous
import jax
import jax.numpy as jnp
import numpy as np
from jax.experimental import pallas as pl
from jax.experimental.pallas import tpu as pltpu

B = 128
V = 100000
N = V - 1

VB = 8192
NVB = (V + VB - 1) // VB
CH = 64
UNROLL = 8
NITER = VB // (UNROLL * CH)

NGB = (N + VB - 1) // VB

_SEED = 42
_KS0 = 0
_KS1 = _SEED
_KS2 = _KS0 ^ _KS1 ^ 0x1BD11BDA
_KS = (_KS0, _KS1, _KS2)
_ROT = ((13, 15, 26, 6), (17, 29, 16, 24))
_TINY = np.float32(np.finfo(np.float32).tiny)
_NEG_INF = np.float32(-np.inf)


def _score(xx, x1):
    u32 = lambda c: jnp.uint32(c & 0xFFFFFFFF)
    x0 = x1
    for i in range(5):
        for j, r in enumerate(_ROT[i % 2]):
            if not (i == 0 and j == 0):
                x0 = x0 + x1
            x1 = (x1 << jnp.uint32(r)) | (x1 >> jnp.uint32(32 - r))
            x1 = x1 ^ x0
        if _KS[(i + 1) % 3]:
            x0 = x0 + u32(_KS[(i + 1) % 3])
        x1 = x1 + u32(_KS[(i + 2) % 3] + (i + 1))
    bits = x0 ^ x1
    f = jax.lax.bitcast_convert_type(
        (bits >> jnp.uint32(9)) | jnp.uint32(0x3F800000), jnp.float32
    ) - jnp.float32(1.0)
    return -jnp.log(-jnp.log(f)) + jnp.log(xx + jnp.float32(1e-20))


def _scan_kernel(xt_ref, c_ref, xo_ref, m_ref, idx_ref):
    step = pl.program_id(0)

    @pl.when(step == 0)
    def _init():
        m_ref[...] = jnp.full((CH, B), _NEG_INF, jnp.float32)
        idx_ref[...] = jnp.zeros((CH, B), jnp.int32)

    sub = jax.lax.broadcasted_iota(jnp.int32, (CH, B), 0)
    lane = jax.lax.broadcasted_iota(jnp.int32, (CH, B), 1)
    base42 = lane * V + sub + (step * VB + 42)
    v0 = step * VB

    def make_body(masked):
        def body(j, carry):
            m, idx = carry
            s0 = j * (UNROLL * CH)
            for k in range(UNROLL):
                sk = s0 + k * CH
                r = _score(xt_ref[pl.ds(sk, CH), :],
                           (base42 + sk).astype(jnp.uint32))
                if masked:
                    r = jnp.where(v0 + sk + sub < V, r, _NEG_INF)
                upd = r > m
                m = jnp.where(upd, r, m)
                idx = jnp.where(upd, step * (UNROLL * NITER) + UNROLL * j + k,
                                idx)
            return m, idx
        return body

    m, idx = jax.lax.cond(
        step == NVB - 1,
        lambda mi: jax.lax.fori_loop(0, NITER, make_body(True), mi),
        lambda mi: jax.lax.fori_loop(0, NITER, make_body(False), mi),
        (m_ref[...], idx_ref[...]),
    )
    m_ref[...] = m
    idx_ref[...] = idx

    xo_ref[...] = xt_ref[...]

    @pl.when(step == NVB - 1)
    def _finish():
        mm = m_ref[...]
        col = idx_ref[...] * CH + sub
        mrow = jnp.max(mm, axis=0, keepdims=True)
        c = jnp.min(jnp.where(mm == mrow, col, jnp.int32(V)),
                    axis=0, keepdims=True)
        c_ref[...] = jnp.broadcast_to(c, (8, B))


def _gate_kernel(c_ref, g_ref):
    step = pl.program_id(0)
    sub = jax.lax.broadcasted_iota(jnp.int32, (VB, B), 0)
    g_ref[...] = (step * VB + sub < c_ref[0:1, :]).astype(jnp.float32)


@jax.jit
def kernel(x):
    xt = x.T
    c8, xot = pl.pallas_call(
        _scan_kernel,
        grid=(NVB,),
        in_specs=[pl.BlockSpec((VB, B), lambda i: (i, 0))],
        out_specs=[
            pl.BlockSpec((8, B), lambda i: (0, 0)),
            pl.BlockSpec((VB, B), lambda i: (i, 0)),
        ],
        out_shape=[
            jax.ShapeDtypeStruct((8, B), jnp.int32),
            jax.ShapeDtypeStruct((V, B), jnp.float32),
        ],
        scratch_shapes=[
            pltpu.VMEM((CH, B), jnp.float32),
            pltpu.VMEM((CH, B), jnp.int32),
        ],
    )(xt)
    gt = pl.pallas_call(
        _gate_kernel,
        grid=(NGB,),
        in_specs=[pl.BlockSpec((8, B), lambda i: (0, 0))],
        out_specs=pl.BlockSpec((VB, B), lambda i: (i, 0)),
        out_shape=jax.ShapeDtypeStruct((N, B), jnp.float32),
    )(c8)
    return (gt.T, xot.T)
